# Initial kernel scaffold; baseline (speedup 1.0000x reference)
import jax
import jax.numpy as jnp
from jax.experimental import pallas as pl


def kernel(z, P, W_M1, b_M1, W_M2, b_M2, W_U, b_U):
    raise NotImplementedError("write your pallas kernel here")


# speedup vs baseline: 1.2731x; 1.2731x over previous
import jax
import jax.numpy as jnp
from jax.experimental import pallas as pl

B, K, Z, H = 4, 256, 128, 128


def _fused_kernel(z_ref, p_ref, w1_ref, b1_ref, w2_ref, b2_ref,
                  wut_ref, wub_ref, bu_ref, out_ref):
    z = z_ref[0]
    m2 = jnp.dot(z, w2_ref[...], preferred_element_type=jnp.float32) + b2_ref[...]
    neg = jnp.float32(-jnp.inf)
    nmT = jnp.where(p_ref[0].T != 0, jnp.float32(0), neg)

    CH = 32
    chunks = []
    for i0 in range(0, K, CH):
        s = m2[None, :, :] + nmT[i0:i0 + CH, :, None]
        chunks.append(jnp.max(s, axis=1))
    M = jnp.concatenate(chunks, axis=0)

    m1 = jnp.dot(z, w1_ref[...], preferred_element_type=jnp.float32) + b1_ref[...]
    m = jnp.where(M == neg, neg, jax.nn.relu(m1 + M))
    acc = jnp.dot(z, wut_ref[...], preferred_element_type=jnp.float32)
    acc = acc + jnp.dot(m, wub_ref[...], preferred_element_type=jnp.float32)
    out_ref[0] = jax.nn.relu(acc + bu_ref[...])


@jax.jit
def kernel(z, P, W_M1, b_M1, W_M2, b_M2, W_U, b_U):
    return pl.pallas_call(
        _fused_kernel,
        grid=(B,),
        in_specs=[
            pl.BlockSpec((1, K, Z), lambda b: (b, 0, 0)),
            pl.BlockSpec((1, K, K), lambda b: (b, 0, 0)),
            pl.BlockSpec((Z, Z), lambda b: (0, 0)),
            pl.BlockSpec((1, Z), lambda b: (0, 0)),
            pl.BlockSpec((Z, Z), lambda b: (0, 0)),
            pl.BlockSpec((1, Z), lambda b: (0, 0)),
            pl.BlockSpec((Z, H), lambda b: (0, 0)),
            pl.BlockSpec((Z, H), lambda b: (0, 0)),
            pl.BlockSpec((1, H), lambda b: (0, 0)),
        ],
        out_specs=pl.BlockSpec((1, K, H), lambda b: (b, 0, 0)),
        out_shape=jax.ShapeDtypeStruct((B, K, H), jnp.float32),
    )(z, P, W_M1, b_M1.reshape(1, Z), W_M2, b_M2.reshape(1, Z),
      W_U[:Z], W_U[Z:], b_U.reshape(1, H))


# speedup vs baseline: 2.1121x; 1.6590x over previous
import jax
import jax.numpy as jnp
from jax.experimental import pallas as pl

B, K, Z, H = 4, 256, 128, 128


def _fused_kernel(z_ref, p_ref, w1_ref, b1_ref, w2_ref, b2_ref,
                  wut_ref, wub_ref, bu_ref, out_ref):
    z = z_ref[0]
    m2 = jnp.dot(z, w2_ref[...], preferred_element_type=jnp.float32) + b2_ref[...]
    neg = jnp.float32(-jnp.inf)
    nm = jnp.where(p_ref[0] != 0, jnp.float32(0), neg)

    rows = []
    for i in range(K):
        s = m2 + nm[:, i:i + 1]
        rows.append(jnp.max(s, axis=0, keepdims=True))
    M = jnp.concatenate(rows, axis=0)

    m1 = jnp.dot(z, w1_ref[...], preferred_element_type=jnp.float32) + b1_ref[...]
    m = jnp.where(M == neg, neg, jax.nn.relu(m1 + M))
    acc = jnp.dot(z, wut_ref[...], preferred_element_type=jnp.float32)
    acc = acc + jnp.dot(m, wub_ref[...], preferred_element_type=jnp.float32)
    out_ref[0] = jax.nn.relu(acc + bu_ref[...])


@jax.jit
def kernel(z, P, W_M1, b_M1, W_M2, b_M2, W_U, b_U):
    return pl.pallas_call(
        _fused_kernel,
        grid=(B,),
        in_specs=[
            pl.BlockSpec((1, K, Z), lambda b: (b, 0, 0)),
            pl.BlockSpec((1, K, K), lambda b: (b, 0, 0)),
            pl.BlockSpec((Z, Z), lambda b: (0, 0)),
            pl.BlockSpec((1, Z), lambda b: (0, 0)),
            pl.BlockSpec((Z, Z), lambda b: (0, 0)),
            pl.BlockSpec((1, Z), lambda b: (0, 0)),
            pl.BlockSpec((Z, H), lambda b: (0, 0)),
            pl.BlockSpec((Z, H), lambda b: (0, 0)),
            pl.BlockSpec((1, H), lambda b: (0, 0)),
        ],
        out_specs=pl.BlockSpec((1, K, H), lambda b: (b, 0, 0)),
        out_shape=jax.ShapeDtypeStruct((B, K, H), jnp.float32),
    )(z, P, W_M1, b_M1.reshape(1, Z), W_M2, b_M2.reshape(1, Z),
      W_U[:Z], W_U[Z:], b_U.reshape(1, H))
